# baseline (speedup 1.0000x reference)
import jax
import jax.numpy as jnp
from jax.experimental import pallas as pl
from jax.experimental.pallas import tpu as pltpu

K_CL = 512
N_TOK = 16384
D = 64
BLK = 512
NBLK = N_TOK // BLK
MAX_ITERS = 100
RTOL = 1e-4
ATOL = 1e-8


def _kernel_body(x_ref, x2_ref, ct0_ref, xa_ref, xb_ref, xc_ref, loss_ref,
                 ct_ref, sums_ref, counts_ref, ids_ref, conv_ref):
    ct_ref[...] = ct0_ref[...]
    conv_ref[0] = 0

    lane_iota = jax.lax.broadcasted_iota(jnp.int32, (BLK, K_CL), 1)
    ones_row = jnp.ones((1, BLK), jnp.bfloat16)

    def iter_body(_, carry):
        @pl.when(conv_ref[0] == 0)
        def _():
            ct = ct_ref[...]
            c2 = jnp.sum(ct * ct, axis=0, keepdims=True)
            sums_ref[...] = jnp.zeros_like(sums_ref)
            counts_ref[...] = jnp.zeros_like(counts_ref)

            def blk_body(b, carry2):
                xb = x_ref[pl.ds(b * BLK, BLK), :]
                x2 = x2_ref[pl.ds(b * BLK, BLK), :]
                dots = jax.lax.dot_general(
                    xb, ct, (((1,), (0,)), ((), ())),
                    preferred_element_type=jnp.float32)
                e = jnp.maximum((x2 + c2) - 2.0 * dots, 0.0)
                emin = jnp.min(e, axis=1, keepdims=True)
                near = (e <= emin * (1.0 + 2.0 ** -18)).astype(jnp.float32)
                ncnt = jnp.sum(near, axis=1, keepdims=True)
                hastie = jnp.max(ncnt) > 1.5

                def _slow_ids():
                    dist = jnp.sqrt(e)
                    dmin = jnp.min(dist, axis=1, keepdims=True)
                    return jnp.min(jnp.where(dist == dmin, lane_iota, K_CL),
                                   axis=1, keepdims=True)

                def _fast_ids():
                    return jnp.min(jnp.where(e == emin, lane_iota, K_CL),
                                   axis=1, keepdims=True)

                idcol = jax.lax.cond(
                    hastie, _slow_ids, _fast_ids).astype(jnp.int32)
                ids_ref[pl.ds(b * BLK, BLK), :] = idcol
                oh = (idcol == lane_iota).astype(jnp.bfloat16)
                sa = jax.lax.dot_general(
                    xa_ref[pl.ds(b * BLK, BLK), :], oh, (((0,), (0,)), ((), ())),
                    preferred_element_type=jnp.float32)
                sb = jax.lax.dot_general(
                    xb_ref[pl.ds(b * BLK, BLK), :], oh, (((0,), (0,)), ((), ())),
                    preferred_element_type=jnp.float32)
                sc = jax.lax.dot_general(
                    xc_ref[pl.ds(b * BLK, BLK), :], oh, (((0,), (0,)), ((), ())),
                    preferred_element_type=jnp.float32)
                sums_ref[...] += (sa + sb) + sc
                counts_ref[...] += jax.lax.dot_general(
                    ones_row, oh, (((1,), (0,)), ((), ())),
                    preferred_element_type=jnp.float32)
                return carry2

            jax.lax.fori_loop(0, NBLK, blk_body, 0, unroll=4)

            cnt = counts_ref[...]
            new_ct = sums_ref[...] / jnp.maximum(cnt, 1.0)
            new_ct = jnp.where(cnt > 0.0, new_ct, ct)
            ac = jnp.all(jnp.abs(ct - new_ct) <= ATOL + RTOL * jnp.abs(new_ct))

            @pl.when(jnp.logical_not(ac))
            def _():
                ct_ref[...] = new_ct

            conv_ref[0] = ac.astype(jnp.int32)

        return carry

    jax.lax.fori_loop(0, MAX_ITERS, iter_body, 0, unroll=False)

    ct = ct_ref[...]

    def loss_blk(b, acc):
        xb = x_ref[pl.ds(b * BLK, BLK), :]
        logits = jax.lax.dot_general(
            xb, ct, (((1,), (0,)), ((), ())),
            preferred_element_type=jnp.float32)
        m = jnp.max(logits, axis=1, keepdims=True)
        lse = m + jnp.log(jnp.sum(jnp.exp(logits - m), axis=1, keepdims=True))
        idcol = ids_ref[pl.ds(b * BLK, BLK), :]
        oh = (idcol == lane_iota).astype(jnp.float32)
        lab = jnp.sum(logits * oh, axis=1, keepdims=True)
        return acc + jnp.sum(lse - lab)

    acc = jax.lax.fori_loop(0, NBLK, loss_blk, jnp.float32(0.0), unroll=False)
    loss_ref[...] = jnp.broadcast_to(acc / jnp.float32(N_TOK), (1, 1))


def _run(x, x2, ct0, xa, xb, xc, interpret=False):
    out = pl.pallas_call(
        _kernel_body,
        out_shape=jax.ShapeDtypeStruct((1, 1), jnp.float32),
        in_specs=[
            pl.BlockSpec(memory_space=pltpu.VMEM),
            pl.BlockSpec(memory_space=pltpu.VMEM),
            pl.BlockSpec(memory_space=pltpu.VMEM),
            pl.BlockSpec(memory_space=pltpu.VMEM),
            pl.BlockSpec(memory_space=pltpu.VMEM),
            pl.BlockSpec(memory_space=pltpu.VMEM),
        ],
        out_specs=pl.BlockSpec(memory_space=pltpu.VMEM),
        scratch_shapes=[
            pltpu.VMEM((D, K_CL), jnp.float32),
            pltpu.VMEM((D, K_CL), jnp.float32),
            pltpu.VMEM((1, K_CL), jnp.float32),
            pltpu.VMEM((N_TOK, 1), jnp.int32),
            pltpu.SMEM((1,), jnp.int32),
        ],
        interpret=interpret,
    )(x, x2, ct0, xa, xb, xc)
    return out[0, 0]


def _split3(x):
    xa = x.astype(jnp.bfloat16)
    r = x - xa.astype(jnp.float32)
    xb = r.astype(jnp.bfloat16)
    xc = (r - xb.astype(jnp.float32)).astype(jnp.bfloat16)
    return xa, xb, xc


def kernel(x):
    perm = jax.random.permutation(jax.random.key(42), N_TOK)
    ct0 = x[perm[:K_CL]].T
    x2 = jnp.sum(x * x, axis=1, keepdims=True)
    xa, xb, xc = _split3(x)
    return _run(x, x2, ct0, xa, xb, xc)


# speedup vs baseline: 1.1159x; 1.1159x over previous
import jax
import jax.numpy as jnp
from jax.experimental import pallas as pl
from jax.experimental.pallas import tpu as pltpu

K_CL = 512
N_TOK = 16384
D = 64
BLK = 512
NBLK = N_TOK // BLK
MAX_ITERS = 100
RTOL = 1e-4
ATOL = 1e-8


def _kernel_body(x_ref, x2_ref, ct0_ref, xa_ref, xb_ref, xc_ref, loss_ref,
                 ct_ref, sums_ref, counts_ref, ids_ref, conv_ref):
    ct_ref[...] = ct0_ref[...]
    conv_ref[0] = 0

    lane_iota = jax.lax.broadcasted_iota(jnp.int32, (BLK, K_CL), 1)
    ones_row = jnp.ones((1, BLK), jnp.bfloat16)

    def iter_body(_, carry):
        @pl.when(conv_ref[0] == 0)
        def _():
            ct = ct_ref[...]
            c2 = jnp.sum(ct * ct, axis=0, keepdims=True)
            ct2 = ct + ct
            sums_ref[...] = jnp.zeros_like(sums_ref)
            counts_ref[...] = jnp.zeros_like(counts_ref)

            def blk_body(b, carry2):
                xb = x_ref[pl.ds(b * BLK, BLK), :]
                x2 = x2_ref[pl.ds(b * BLK, BLK), :]
                dots2 = jax.lax.dot_general(
                    xb, ct2, (((1,), (0,)), ((), ())),
                    preferred_element_type=jnp.float32)
                e = jnp.maximum((x2 + c2) - dots2, 0.0)
                emin = jnp.min(e, axis=1, keepdims=True)
                near = (e <= emin * (1.0 + 2.0 ** -18)).astype(jnp.float32)
                ncnt = jnp.sum(near, axis=1, keepdims=True)
                idcol = jnp.min(jnp.where(e == emin, lane_iota, K_CL),
                                axis=1, keepdims=True).astype(jnp.int32)
                ids_ref[pl.ds(b * BLK, BLK), :] = idcol

                @pl.when(jnp.max(ncnt) > 1.5)
                def _():
                    dist = jnp.sqrt(e)
                    dmin = jnp.min(dist, axis=1, keepdims=True)
                    ids_ref[pl.ds(b * BLK, BLK), :] = jnp.min(
                        jnp.where(dist == dmin, lane_iota, K_CL),
                        axis=1, keepdims=True).astype(jnp.int32)

                idcol = ids_ref[pl.ds(b * BLK, BLK), :]
                oh = (idcol == lane_iota).astype(jnp.bfloat16)
                sa = jax.lax.dot_general(
                    xa_ref[pl.ds(b * BLK, BLK), :], oh, (((0,), (0,)), ((), ())),
                    preferred_element_type=jnp.float32)
                sb = jax.lax.dot_general(
                    xb_ref[pl.ds(b * BLK, BLK), :], oh, (((0,), (0,)), ((), ())),
                    preferred_element_type=jnp.float32)
                sc = jax.lax.dot_general(
                    xc_ref[pl.ds(b * BLK, BLK), :], oh, (((0,), (0,)), ((), ())),
                    preferred_element_type=jnp.float32)
                sums_ref[...] += (sa + sb) + sc
                counts_ref[...] += jax.lax.dot_general(
                    ones_row, oh, (((1,), (0,)), ((), ())),
                    preferred_element_type=jnp.float32)
                return carry2

            jax.lax.fori_loop(0, NBLK, blk_body, 0, unroll=4)

            cnt = counts_ref[...]
            new_ct = sums_ref[...] / jnp.maximum(cnt, 1.0)
            new_ct = jnp.where(cnt > 0.0, new_ct, ct)
            ac = jnp.all(jnp.abs(ct - new_ct) <= ATOL + RTOL * jnp.abs(new_ct))

            @pl.when(jnp.logical_not(ac))
            def _():
                ct_ref[...] = new_ct

            conv_ref[0] = ac.astype(jnp.int32)

        return carry

    jax.lax.fori_loop(0, MAX_ITERS, iter_body, 0, unroll=False)

    ct = ct_ref[...]

    def loss_blk(b, acc):
        xb = x_ref[pl.ds(b * BLK, BLK), :]
        logits = jax.lax.dot_general(
            xb, ct, (((1,), (0,)), ((), ())),
            preferred_element_type=jnp.float32)
        m = jnp.max(logits, axis=1, keepdims=True)
        lse = m + jnp.log(jnp.sum(jnp.exp(logits - m), axis=1, keepdims=True))
        idcol = ids_ref[pl.ds(b * BLK, BLK), :]
        oh = (idcol == lane_iota).astype(jnp.float32)
        lab = jnp.sum(logits * oh, axis=1, keepdims=True)
        return acc + jnp.sum(lse - lab)

    acc = jax.lax.fori_loop(0, NBLK, loss_blk, jnp.float32(0.0), unroll=False)
    loss_ref[...] = jnp.broadcast_to(acc / jnp.float32(N_TOK), (1, 1))


def _run(x, x2, ct0, xa, xb, xc, interpret=False):
    out = pl.pallas_call(
        _kernel_body,
        out_shape=jax.ShapeDtypeStruct((1, 1), jnp.float32),
        in_specs=[
            pl.BlockSpec(memory_space=pltpu.VMEM),
            pl.BlockSpec(memory_space=pltpu.VMEM),
            pl.BlockSpec(memory_space=pltpu.VMEM),
            pl.BlockSpec(memory_space=pltpu.VMEM),
            pl.BlockSpec(memory_space=pltpu.VMEM),
            pl.BlockSpec(memory_space=pltpu.VMEM),
        ],
        out_specs=pl.BlockSpec(memory_space=pltpu.VMEM),
        scratch_shapes=[
            pltpu.VMEM((D, K_CL), jnp.float32),
            pltpu.VMEM((D, K_CL), jnp.float32),
            pltpu.VMEM((1, K_CL), jnp.float32),
            pltpu.VMEM((N_TOK, 1), jnp.int32),
            pltpu.SMEM((1,), jnp.int32),
        ],
        interpret=interpret,
    )(x, x2, ct0, xa, xb, xc)
    return out[0, 0]


def _split3(x):
    xa = x.astype(jnp.bfloat16)
    r = x - xa.astype(jnp.float32)
    xb = r.astype(jnp.bfloat16)
    xc = (r - xb.astype(jnp.float32)).astype(jnp.bfloat16)
    return xa, xb, xc


def kernel(x):
    perm = jax.random.permutation(jax.random.key(42), N_TOK)
    ct0 = x[perm[:K_CL]].T
    x2 = jnp.sum(x * x, axis=1, keepdims=True)
    xa, xb, xc = _split3(x)
    return _run(x, x2, ct0, xa, xb, xc)


# speedup vs baseline: 1.1415x; 1.0230x over previous
import jax
import jax.numpy as jnp
from jax.experimental import pallas as pl
from jax.experimental.pallas import tpu as pltpu

K_CL = 512
N_TOK = 16384
D = 64
BLK = 512
NBLK = N_TOK // BLK
MAX_ITERS = 100
RTOL = 1e-4
ATOL = 1e-8


def _kernel_body(x_ref, x2_ref, ct0_ref, xa_ref, xb_ref, xc_ref, loss_ref,
                 ct_ref, sums_ref, counts_ref, ids_ref, conv_ref):
    ct_ref[...] = ct0_ref[...]
    conv_ref[0] = 0

    lane_iota = jax.lax.broadcasted_iota(jnp.int32, (BLK, K_CL), 1)
    ones_row = jnp.ones((1, BLK), jnp.bfloat16)

    def iter_body(_, carry):
        @pl.when(conv_ref[0] == 0)
        def _():
            ct = ct_ref[...]
            c2 = jnp.sum(ct * ct, axis=0, keepdims=True)
            ct2 = ct + ct
            sums_ref[...] = jnp.zeros_like(sums_ref)
            counts_ref[...] = jnp.zeros_like(counts_ref)

            def blk_body(b, carry2):
                xb = x_ref[pl.ds(b * BLK, BLK), :]
                x2 = x2_ref[pl.ds(b * BLK, BLK), :]
                dots2 = jax.lax.dot_general(
                    xb, ct2, (((1,), (0,)), ((), ())),
                    preferred_element_type=jnp.float32)
                dist = jnp.sqrt(jnp.maximum((x2 + c2) - dots2, 0.0))
                dmin = jnp.min(dist, axis=1, keepdims=True)
                idcol = jnp.min(
                    jnp.where(dist == dmin, lane_iota, K_CL),
                    axis=1, keepdims=True).astype(jnp.int32)
                ids_ref[pl.ds(b * BLK, BLK), :] = idcol
                oh = (idcol == lane_iota).astype(jnp.bfloat16)
                sa = jax.lax.dot_general(
                    xa_ref[pl.ds(b * BLK, BLK), :], oh, (((0,), (0,)), ((), ())),
                    preferred_element_type=jnp.float32)
                sb = jax.lax.dot_general(
                    xb_ref[pl.ds(b * BLK, BLK), :], oh, (((0,), (0,)), ((), ())),
                    preferred_element_type=jnp.float32)
                sc = jax.lax.dot_general(
                    xc_ref[pl.ds(b * BLK, BLK), :], oh, (((0,), (0,)), ((), ())),
                    preferred_element_type=jnp.float32)
                sums_ref[...] += (sa + sb) + sc
                counts_ref[...] += jax.lax.dot_general(
                    ones_row, oh, (((1,), (0,)), ((), ())),
                    preferred_element_type=jnp.float32)
                return carry2

            jax.lax.fori_loop(0, NBLK, blk_body, 0, unroll=4)

            cnt = counts_ref[...]
            new_ct = sums_ref[...] / jnp.maximum(cnt, 1.0)
            new_ct = jnp.where(cnt > 0.0, new_ct, ct)
            ac = jnp.all(jnp.abs(ct - new_ct) <= ATOL + RTOL * jnp.abs(new_ct))

            @pl.when(jnp.logical_not(ac))
            def _():
                ct_ref[...] = new_ct

            conv_ref[0] = ac.astype(jnp.int32)

        return carry

    jax.lax.fori_loop(0, MAX_ITERS, iter_body, 0, unroll=False)

    ct = ct_ref[...]

    def loss_blk(b, acc):
        xb = x_ref[pl.ds(b * BLK, BLK), :]
        logits = jax.lax.dot_general(
            xb, ct, (((1,), (0,)), ((), ())),
            preferred_element_type=jnp.float32)
        m = jnp.max(logits, axis=1, keepdims=True)
        lse = m + jnp.log(jnp.sum(jnp.exp(logits - m), axis=1, keepdims=True))
        idcol = ids_ref[pl.ds(b * BLK, BLK), :]
        oh = (idcol == lane_iota).astype(jnp.float32)
        lab = jnp.sum(logits * oh, axis=1, keepdims=True)
        return acc + jnp.sum(lse - lab)

    acc = jax.lax.fori_loop(0, NBLK, loss_blk, jnp.float32(0.0), unroll=False)
    loss_ref[...] = jnp.broadcast_to(acc / jnp.float32(N_TOK), (1, 1))


def _run(x, x2, ct0, xa, xb, xc, interpret=False):
    out = pl.pallas_call(
        _kernel_body,
        out_shape=jax.ShapeDtypeStruct((1, 1), jnp.float32),
        in_specs=[
            pl.BlockSpec(memory_space=pltpu.VMEM),
            pl.BlockSpec(memory_space=pltpu.VMEM),
            pl.BlockSpec(memory_space=pltpu.VMEM),
            pl.BlockSpec(memory_space=pltpu.VMEM),
            pl.BlockSpec(memory_space=pltpu.VMEM),
            pl.BlockSpec(memory_space=pltpu.VMEM),
        ],
        out_specs=pl.BlockSpec(memory_space=pltpu.VMEM),
        scratch_shapes=[
            pltpu.VMEM((D, K_CL), jnp.float32),
            pltpu.VMEM((D, K_CL), jnp.float32),
            pltpu.VMEM((1, K_CL), jnp.float32),
            pltpu.VMEM((N_TOK, 1), jnp.int32),
            pltpu.SMEM((1,), jnp.int32),
        ],
        interpret=interpret,
    )(x, x2, ct0, xa, xb, xc)
    return out[0, 0]


def _split3(x):
    xa = x.astype(jnp.bfloat16)
    r = x - xa.astype(jnp.float32)
    xb = r.astype(jnp.bfloat16)
    xc = (r - xb.astype(jnp.float32)).astype(jnp.bfloat16)
    return xa, xb, xc


def kernel(x):
    perm = jax.random.permutation(jax.random.key(42), N_TOK)
    ct0 = x[perm[:K_CL]].T
    x2 = jnp.sum(x * x, axis=1, keepdims=True)
    xa, xb, xc = _split3(x)
    return _run(x, x2, ct0, xa, xb, xc)
